# baseline (speedup 1.0000x reference)
import functools

import jax
import jax.numpy as jnp
from jax import lax
from jax.experimental import pallas as pl
from jax.experimental.pallas import tpu as pltpu
from jax.experimental.pallas import tpu_sc as plsc

_V = 1000000
_D = 64
_B = 16384
_NC = 2
_NS = 16
_NW = _NC * _NS
_CHUNK = 256
_CSHIFT = 8
_NCHUNKS = _V // _CHUNK
_TAIL_LO = _NCHUNKS * _CHUNK
_TAIL_N = _V - _TAIL_LO
_DUMP = _B
_SENT = 0x7FFFFFFF


def _scatter_half(stage, posb, out_hbm, sem_out, half):
    base = pl.multiple_of(half * 128, 128)
    return pltpu.async_copy(
        stage.at[pl.ds(base, 128)], out_hbm.at[posb.at[half]], sem_out
    )


def _drain_one(stage, posb, out_hbm, sem_out):
    pltpu.make_async_copy(
        stage.at[pl.ds(0, 128)], out_hbm.at[posb.at[0]], sem_out
    ).wait()


def _process(chunk_ref, clo, width, n_groups, r0, refs):
    mv, mp, stage, posb, out_hbm, sem_out, iota = refs
    chi = clo + width

    def grp(g, r):
        v = mv[pl.ds(g * 16, 16)]
        m = (v >= clo) & (v < chi)
        pc = jnp.sum(m.astype(jnp.int32))
        rn = r + pc

        @pl.when(pc > 0)
        def _():
            p = mp[pl.ds(g * 16, 16)]
            cols = v - clo
            rank = plsc.cumsum(m.astype(jnp.int32)) - 1
            rowidx = (r + rank) & 255
            for d in range(_D):
                dvec = jnp.full((16,), d, jnp.int32)
                val = plsc.load_gather(chunk_ref, [dvec, cols], mask=m)
                plsc.store_scatter(stage, [rowidx, dvec], val, mask=m)
            plsc.store_scatter(
                posb, [lax.shift_right_logical(rowidx, 7), rowidx & 127],
                p, mask=m)

        @pl.when(lax.shift_right_logical(rn, 7)
                 > lax.shift_right_logical(r, 7))
        def _():
            s_old = lax.shift_right_logical(r, 7)

            @pl.when(s_old >= 1)
            def _():
                _drain_one(stage, posb, out_hbm, sem_out)

            _scatter_half(stage, posb, out_hbm, sem_out, s_old & 1)

        return rn

    return lax.fori_loop(0, n_groups, grp, r0)


@functools.lru_cache(maxsize=None)
def _build():
    mesh = plsc.VectorSubcoreMesh(core_axis_name="c", subcore_axis_name="s")

    @functools.partial(
        pl.kernel,
        mesh=mesh,
        out_type=jax.ShapeDtypeStruct((_B + 16, 128), jnp.float32),
        scratch_types=[
            pltpu.VMEM((_B,), jnp.int32),
            pltpu.VMEM((_B + 16,), jnp.int32),
            pltpu.VMEM((_B + 16,), jnp.int32),
            pltpu.VMEM((_D, _CHUNK), jnp.float32),
            pltpu.VMEM((_D, _CHUNK), jnp.float32),
            pltpu.VMEM((_D, _TAIL_N), jnp.float32),
            pltpu.VMEM((256, 128), jnp.float32),
            pltpu.VMEM((2, 128), jnp.int32),
            pltpu.SemaphoreType.DMA,
            pltpu.SemaphoreType.DMA,
            pltpu.SemaphoreType.DMA,
        ],
        compiler_params=pltpu.CompilerParams(
            use_tc_tiling_on_sc=True, needs_layout_passes=False
        ),
    )
    def gather_kernel(tableT, tail_hbm, idx_hbm, out_hbm,
                      idx_v, mv, mp, chunk0, chunk1, tail_v, stage, posb,
                      sem0, sem1, sem_out):
        wid = lax.axis_index("s") * _NC + lax.axis_index("c")
        iota = lax.iota(jnp.int32, 16)
        nk = jnp.where(wid < _NCHUNKS % _NW, _NCHUNKS // _NW + 1,
                       _NCHUNKS // _NW)
        chunks = (chunk0, chunk1)
        sems = (sem0, sem1)

        def chunk_slice(k):
            off = pl.multiple_of((wid + k * _NW) * _CHUNK, 128)
            return tableT.at[:, pl.ds(off, _CHUNK)]

        pltpu.async_copy(chunk_slice(0), chunk0, sem0)

        pltpu.sync_copy(idx_hbm, idx_v)

        def route(i, off):
            v = idx_v[pl.ds(i * 16, 16)]
            c = lax.shift_right_logical(v, _CSHIFT)
            m = ((c & (_NW - 1)) == wid) & (v < _TAIL_LO)
            m = m | ((v >= _TAIL_LO) & (v < _V) & (wid == _NW - 1))
            plsc.store_compressed(mv.at[pl.ds(off, 16)], v, mask=m)
            plsc.store_compressed(mp.at[pl.ds(off, 16)], iota + i * 16,
                                  mask=m)
            return off + jnp.sum(m.astype(jnp.int32))

        off = lax.fori_loop(0, _B // 16, route, jnp.int32(0))
        mv[pl.ds(off, 16)] = jnp.full((16,), _SENT, jnp.int32)
        mp[pl.ds(off, 16)] = jnp.full((16,), _DUMP, jnp.int32)
        n_groups = lax.shift_right_logical(off, 4) + 1

        refs = (mv, mp, stage, posb, out_hbm, sem_out, iota)

        def pair(j, r):
            for b in range(2):
                k = 2 * j + b
                nxt = k + 1

                @pl.when(nxt < nk)
                def _():
                    pltpu.async_copy(chunk_slice(nxt), chunks[1 - b],
                                     sems[1 - b])

                @pl.when(k < nk)
                def _():
                    pltpu.make_async_copy(chunk_slice(k), chunks[b],
                                          sems[b]).wait()

                clo = jnp.where(k < nk, (wid + k * _NW) * _CHUNK,
                                jnp.int32(2 ** 30))
                r = _process(chunks[b], clo, _CHUNK, n_groups, r, refs)
            return r

        r = lax.fori_loop(0, (_NCHUNKS // _NW + 2) // 2, pair, jnp.int32(0))

        @pl.when(wid == _NW - 1)
        def _():
            pltpu.sync_copy(tail_hbm, tail_v)

        tclo = jnp.where(wid == _NW - 1, jnp.int32(_TAIL_LO),
                         jnp.int32(2 ** 30))
        r = _process(tail_v, tclo, _TAIL_N, n_groups, r, refs)

        rrem = r & 127
        s_done = lax.shift_right_logical(r, 7)

        @pl.when(rrem > 0)
        def _():
            half = s_done & 1
            for j in range(8):
                colv = iota + j * 16
                old = posb[half, pl.ds(j * 16, 16)]
                posb[half, pl.ds(j * 16, 16)] = jnp.where(
                    colv < rrem, old, _DUMP)

            @pl.when(s_done >= 1)
            def _():
                _drain_one(stage, posb, out_hbm, sem_out)

            _scatter_half(stage, posb, out_hbm, sem_out, half)
            _drain_one(stage, posb, out_hbm, sem_out)

        @pl.when((rrem == 0) & (s_done >= 1))
        def _():
            _drain_one(stage, posb, out_hbm, sem_out)

    return gather_kernel


def kernel(user_embeddings, batch_data):
    tt = user_embeddings.T
    tail = tt[:, _TAIL_LO:]
    idx = batch_data.astype(jnp.int32)
    out = _build()(tt, tail, idx)
    return out[:_B, :_D]


# speedup vs baseline: 1.4028x; 1.4028x over previous
import functools

import jax
import jax.numpy as jnp
from jax import lax
from jax.experimental import pallas as pl
from jax.experimental.pallas import tpu as pltpu
from jax.experimental.pallas import tpu_sc as plsc

_V = 1000000
_D = 64
_B = 16384
_NC = 2
_NS = 16
_NW = _NC * _NS
_CHUNK = 512
_CSHIFT = 9
_NCHUNKS = _V // _CHUNK
_TAIL_LO = _NCHUNKS * _CHUNK
_TAIL_N = _V - _TAIL_LO
_DUMP = _B
_SENT = 0x7FFFFFFF


def _scatter_half(stage, posb, out_hbm, sem_out, half):
    base = pl.multiple_of(half * 64, 64)
    return pltpu.async_copy(
        stage.at[pl.ds(base, 64)], out_hbm.at[posb.at[half]], sem_out
    )


def _drain_one(stage, posb, out_hbm, sem_out):
    pltpu.make_async_copy(
        stage.at[pl.ds(0, 64)], out_hbm.at[posb.at[0]], sem_out
    ).wait()


def _process(chunk_ref, clo, width, n_groups, r0, refs):
    idx_v, mp, stage, posb, out_hbm, sem_out, iota = refs
    chi = clo + width

    def grp(g, r):
        p = mp[pl.ds(g * 16, 16)]
        v = plsc.load_gather(idx_v, [p])
        m = (v >= clo) & (v < chi)
        pc = jnp.sum(m.astype(jnp.int32))
        rn = r + pc

        @pl.when(pc > 0)
        def _():
            cols = v - clo
            rank = plsc.cumsum(m.astype(jnp.int32)) - 1
            rowidx = (r + rank) & 127
            for d in range(_D):
                dvec = jnp.full((16,), d, jnp.int32)
                val = plsc.load_gather(chunk_ref, [dvec, cols], mask=m)
                plsc.store_scatter(stage, [rowidx, dvec], val, mask=m)
            plsc.store_scatter(
                posb, [lax.shift_right_logical(rowidx, 6), rowidx & 63],
                p, mask=m)

        @pl.when(lax.shift_right_logical(rn, 6)
                 > lax.shift_right_logical(r, 6))
        def _():
            s_old = lax.shift_right_logical(r, 6)

            @pl.when(s_old >= 1)
            def _():
                _drain_one(stage, posb, out_hbm, sem_out)

            _scatter_half(stage, posb, out_hbm, sem_out, s_old & 1)

        return rn

    return lax.fori_loop(0, n_groups, grp, r0)


@functools.lru_cache(maxsize=None)
def _build():
    mesh = plsc.VectorSubcoreMesh(core_axis_name="c", subcore_axis_name="s")

    @functools.partial(
        pl.kernel,
        mesh=mesh,
        out_type=jax.ShapeDtypeStruct((_B + 16, 128), jnp.float32),
        scratch_types=[
            pltpu.VMEM((_B + 16,), jnp.int32),
            pltpu.VMEM((_B + 16,), jnp.int32),
            pltpu.VMEM((_D, _CHUNK), jnp.float32),
            pltpu.VMEM((_D, _CHUNK), jnp.float32),
            pltpu.VMEM((_D, _TAIL_N), jnp.float32),
            pltpu.VMEM((128, 128), jnp.float32),
            pltpu.VMEM((2, 64), jnp.int32),
            pltpu.SemaphoreType.DMA,
            pltpu.SemaphoreType.DMA,
            pltpu.SemaphoreType.DMA,
        ],
        compiler_params=pltpu.CompilerParams(
            use_tc_tiling_on_sc=True, needs_layout_passes=False
        ),
    )
    def gather_kernel(tableT, tail_hbm, idx_hbm, out_hbm,
                      idx_v, mp, chunk0, chunk1, tail_v, stage, posb,
                      sem0, sem1, sem_out):
        wid = lax.axis_index("s") * _NC + lax.axis_index("c")
        iota = lax.iota(jnp.int32, 16)
        nk = jnp.where(wid < _NCHUNKS % _NW, _NCHUNKS // _NW + 1,
                       _NCHUNKS // _NW)
        chunks = (chunk0, chunk1)
        sems = (sem0, sem1)

        def chunk_slice(k):
            off = pl.multiple_of((wid + k * _NW) * _CHUNK, 128)
            return tableT.at[:, pl.ds(off, _CHUNK)]

        pltpu.async_copy(chunk_slice(0), chunk0, sem0)

        pltpu.sync_copy(idx_hbm, idx_v.at[pl.ds(0, _B)])
        idx_v[pl.ds(_B, 16)] = jnp.full((16,), _SENT, jnp.int32)

        def route(i, off):
            v = idx_v[pl.ds(i * 16, 16)]
            c = lax.shift_right_logical(v, _CSHIFT)
            m = ((c & (_NW - 1)) == wid) & (v < _TAIL_LO)
            m = m | ((v >= _TAIL_LO) & (v < _V) & (wid == _NW - 1))
            plsc.store_compressed(mp.at[pl.ds(off, 16)], iota + i * 16,
                                  mask=m)
            return off + jnp.sum(m.astype(jnp.int32))

        off = lax.fori_loop(0, _B // 16, route, jnp.int32(0))
        mp[pl.ds(off, 16)] = jnp.full((16,), _B, jnp.int32)
        n_groups = lax.shift_right_logical(off, 4) + 1

        refs = (idx_v, mp, stage, posb, out_hbm, sem_out, iota)

        def pair(j, r):
            for b in range(2):
                k = 2 * j + b
                nxt = k + 1

                @pl.when(nxt < nk)
                def _():
                    pltpu.async_copy(chunk_slice(nxt), chunks[1 - b],
                                     sems[1 - b])

                @pl.when(k < nk)
                def _():
                    pltpu.make_async_copy(chunk_slice(k), chunks[b],
                                          sems[b]).wait()

                clo = jnp.where(k < nk, (wid + k * _NW) * _CHUNK,
                                jnp.int32(2 ** 30))
                r = _process(chunks[b], clo, _CHUNK, n_groups, r, refs)
            return r

        r = lax.fori_loop(0, (_NCHUNKS // _NW + 2) // 2, pair, jnp.int32(0))

        @pl.when(wid == _NW - 1)
        def _():
            pltpu.sync_copy(tail_hbm, tail_v)

        tclo = jnp.where(wid == _NW - 1, jnp.int32(_TAIL_LO),
                         jnp.int32(2 ** 30))
        r = _process(tail_v, tclo, _TAIL_N, n_groups, r, refs)

        rrem = r & 63
        s_done = lax.shift_right_logical(r, 6)

        @pl.when(rrem > 0)
        def _():
            half = s_done & 1
            for j in range(4):
                colv = iota + j * 16
                old = posb[half, pl.ds(j * 16, 16)]
                posb[half, pl.ds(j * 16, 16)] = jnp.where(
                    colv < rrem, old, _DUMP)

            @pl.when(s_done >= 1)
            def _():
                _drain_one(stage, posb, out_hbm, sem_out)

            _scatter_half(stage, posb, out_hbm, sem_out, half)
            _drain_one(stage, posb, out_hbm, sem_out)

        @pl.when((rrem == 0) & (s_done >= 1))
        def _():
            _drain_one(stage, posb, out_hbm, sem_out)

    return gather_kernel


def kernel(user_embeddings, batch_data):
    tt = user_embeddings.T
    tail = tt[:, _TAIL_LO:]
    idx = batch_data.astype(jnp.int32)
    out = _build()(tt, tail, idx)
    return out[:_B, :_D]


# speedup vs baseline: 1.9067x; 1.3592x over previous
import functools

import jax
import jax.numpy as jnp
from jax import lax
from jax.experimental import pallas as pl
from jax.experimental.pallas import tpu as pltpu
from jax.experimental.pallas import tpu_sc as plsc

_V = 1000000
_D = 64
_B = 16384
_NC = 2
_NS = 16
_NW = _NC * _NS
_CHUNK = 512
_CSHIFT = 9
_NCHUNKS = _V // _CHUNK
_TAIL_LO = _NCHUNKS * _CHUNK
_TAIL_N = _V - _TAIL_LO
_DUMP = _B
_SENT = 0x7FFFFFFF


def _scatter_half(stage, posb, out_hbm, sem_out, half):
    base = pl.multiple_of(half * 64, 64)
    return pltpu.async_copy(
        stage.at[pl.ds(base, 64)], out_hbm.at[posb.at[half]], sem_out
    )


def _drain_one(stage, posb, out_hbm, sem_out):
    pltpu.make_async_copy(
        stage.at[pl.ds(0, 64)], out_hbm.at[posb.at[0]], sem_out
    ).wait()


def _append(chunk_ref, cols, pos, m, r, pc, refs):
    idx_v, mp, pend_c, pend_p, stage, posb, out_hbm, sem_out, iota = refs
    rowidx = (r + iota) & 127
    for d in range(_D):
        dvec = jnp.full((16,), d, jnp.int32)
        val = plsc.load_gather(chunk_ref, [dvec, cols], mask=m)
        plsc.store_scatter(stage, [rowidx, dvec], val, mask=m)
    plsc.store_scatter(
        posb, [lax.shift_right_logical(rowidx, 6), rowidx & 63],
        pos, mask=m)

    @pl.when(lax.shift_right_logical(r + pc, 6)
             > lax.shift_right_logical(r, 6))
    def _():
        s_old = lax.shift_right_logical(r, 6)

        @pl.when(s_old >= 1)
        def _():
            _drain_one(stage, posb, out_hbm, sem_out)

        _scatter_half(stage, posb, out_hbm, sem_out, s_old & 1)


def _process(chunk_ref, clo, width, n_groups, r0, refs):
    idx_v, mp, pend_c, pend_p, stage, posb, out_hbm, sem_out, iota = refs
    chi = clo + width
    full = iota < 16

    def grp(g, carry):
        cur, r = carry
        p = mp[pl.ds(g * 16, 16)]
        v = idx_v[pl.ds(g * 16, 16)]
        m = (v >= clo) & (v < chi)
        pc = jnp.sum(m.astype(jnp.int32))

        @pl.when(pc > 0)
        def _():
            plsc.store_compressed(pend_c.at[pl.ds(cur, 16)], v - clo,
                                  mask=m)
            plsc.store_compressed(pend_p.at[pl.ds(cur, 16)], p, mask=m)

        ncur = cur + pc
        do_flush = ncur >= 16

        @pl.when(do_flush)
        def _():
            _append(chunk_ref, pend_c[pl.ds(0, 16)], pend_p[pl.ds(0, 16)],
                    full, r, jnp.int32(16), refs)
            pend_c[pl.ds(0, 16)] = pend_c[pl.ds(16, 16)]
            pend_p[pl.ds(0, 16)] = pend_p[pl.ds(16, 16)]

        return (jnp.where(do_flush, ncur - 16, ncur),
                jnp.where(do_flush, r + 16, r))

    cur, r = lax.fori_loop(0, n_groups, grp, (jnp.int32(0), r0))

    @pl.when(cur > 0)
    def _():
        _append(chunk_ref, pend_c[pl.ds(0, 16)], pend_p[pl.ds(0, 16)],
                iota < cur, r, cur, refs)

    return r + cur


@functools.lru_cache(maxsize=None)
def _build():
    mesh = plsc.VectorSubcoreMesh(core_axis_name="c", subcore_axis_name="s")

    @functools.partial(
        pl.kernel,
        mesh=mesh,
        out_type=jax.ShapeDtypeStruct((_B + 16, 128), jnp.float32),
        scratch_types=[
            pltpu.VMEM((_B + 16,), jnp.int32),
            pltpu.VMEM((_B + 16,), jnp.int32),
            pltpu.VMEM((_D, _CHUNK), jnp.float32),
            pltpu.VMEM((_D, _CHUNK), jnp.float32),
            pltpu.VMEM((_D, _TAIL_N), jnp.float32),
            pltpu.VMEM((128, 128), jnp.float32),
            pltpu.VMEM((2, 64), jnp.int32),
            pltpu.VMEM((32,), jnp.int32),
            pltpu.VMEM((32,), jnp.int32),
            pltpu.SemaphoreType.DMA,
            pltpu.SemaphoreType.DMA,
            pltpu.SemaphoreType.DMA,
        ],
        compiler_params=pltpu.CompilerParams(
            use_tc_tiling_on_sc=True, needs_layout_passes=False
        ),
    )
    def gather_kernel(tableT, tail_hbm, idx_hbm, out_hbm,
                      idx_v, mp, chunk0, chunk1, tail_v, stage, posb,
                      pend_c, pend_p, sem0, sem1, sem_out):
        wid = lax.axis_index("s") * _NC + lax.axis_index("c")
        iota = lax.iota(jnp.int32, 16)
        nk = jnp.where(wid < _NCHUNKS % _NW, _NCHUNKS // _NW + 1,
                       _NCHUNKS // _NW)
        chunks = (chunk0, chunk1)
        sems = (sem0, sem1)

        def chunk_slice(k):
            off = pl.multiple_of((wid + k * _NW) * _CHUNK, 128)
            return tableT.at[:, pl.ds(off, _CHUNK)]

        pltpu.async_copy(chunk_slice(0), chunk0, sem0)

        pltpu.sync_copy(idx_hbm, idx_v.at[pl.ds(0, _B)])

        def route(i, off):
            v = idx_v[pl.ds(i * 16, 16)]
            c = lax.shift_right_logical(v, _CSHIFT)
            m = ((c & (_NW - 1)) == wid) & (v < _TAIL_LO)
            m = m | ((v >= _TAIL_LO) & (v < _V) & (wid == _NW - 1))
            plsc.store_compressed(mp.at[pl.ds(off, 16)], iota + i * 16,
                                  mask=m)
            plsc.store_compressed(idx_v.at[pl.ds(off, 16)], v, mask=m)
            return off + jnp.sum(m.astype(jnp.int32))

        off = lax.fori_loop(0, _B // 16, route, jnp.int32(0))
        mp[pl.ds(off, 16)] = jnp.full((16,), _B, jnp.int32)
        idx_v[pl.ds(off, 16)] = jnp.full((16,), _SENT, jnp.int32)
        n_groups = lax.shift_right_logical(off, 4) + 1

        refs = (idx_v, mp, pend_c, pend_p, stage, posb, out_hbm,
                sem_out, iota)

        def pair(j, r):
            for b in range(2):
                k = 2 * j + b
                nxt = k + 1

                @pl.when(nxt < nk)
                def _():
                    pltpu.async_copy(chunk_slice(nxt), chunks[1 - b],
                                     sems[1 - b])

                @pl.when(k < nk)
                def _():
                    pltpu.make_async_copy(chunk_slice(k), chunks[b],
                                          sems[b]).wait()

                clo = jnp.where(k < nk, (wid + k * _NW) * _CHUNK,
                                jnp.int32(2 ** 30))
                r = _process(chunks[b], clo, _CHUNK, n_groups, r, refs)
            return r

        r = lax.fori_loop(0, (_NCHUNKS // _NW + 2) // 2, pair, jnp.int32(0))

        @pl.when(wid == _NW - 1)
        def _():
            pltpu.sync_copy(tail_hbm, tail_v)

        tclo = jnp.where(wid == _NW - 1, jnp.int32(_TAIL_LO),
                         jnp.int32(2 ** 30))
        r = _process(tail_v, tclo, _TAIL_N, n_groups, r, refs)

        rrem = r & 63
        s_done = lax.shift_right_logical(r, 6)

        @pl.when(rrem > 0)
        def _():
            half = s_done & 1
            for j in range(4):
                colv = iota + j * 16
                old = posb[half, pl.ds(j * 16, 16)]
                posb[half, pl.ds(j * 16, 16)] = jnp.where(
                    colv < rrem, old, _DUMP)

            @pl.when(s_done >= 1)
            def _():
                _drain_one(stage, posb, out_hbm, sem_out)

            _scatter_half(stage, posb, out_hbm, sem_out, half)
            _drain_one(stage, posb, out_hbm, sem_out)

        @pl.when((rrem == 0) & (s_done >= 1))
        def _():
            _drain_one(stage, posb, out_hbm, sem_out)

    return gather_kernel


def kernel(user_embeddings, batch_data):
    tt = user_embeddings.T
    tail = tt[:, _TAIL_LO:]
    idx = batch_data.astype(jnp.int32)
    out = _build()(tt, tail, idx)
    return out[:_B, :_D]
